# baseline (speedup 1.0000x reference)
import functools

import jax
import jax.numpy as jnp
from jax.experimental import pallas as pl
from jax.experimental.pallas import tpu as pltpu


def _compress_kernel(k_ref, v_ref, knw_ref, vnw_ref, ck_ref, cv_ref,
                     *, NB, BS, D):
    hk = pl.program_id(0)
    kb = k_ref[hk].reshape(NB, BS, D).mean(axis=1)
    vb = v_ref[hk].reshape(NB, BS, D).mean(axis=1)
    ck_ref[0] = kb * jax.lax.rsqrt(
        jnp.mean(kb * kb, axis=-1, keepdims=True) + 1e-6) * knw_ref[...]
    cv_ref[0] = vb * jax.lax.rsqrt(
        jnp.mean(vb * vb, axis=-1, keepdims=True) + 1e-6) * vnw_ref[...]


def _fused_kernel(q_ref, k_ref, v_ref, ck_ref, cv_ref, memk_ref, memv_ref,
                  scw_ref, scb_ref, hid_ref, chw_ref, exp_ref, prev_ref,
                  out_ref,
                  *, QB, S, HK, G, D, BS, NB, NUM_SEL, WIN, scale, QOFF):
    qi = pl.program_id(0)
    base = (QOFF + qi) * QB
    BIG = jnp.float32(2.0 ** 30)
    row_s = base + jax.lax.broadcasted_iota(jnp.int32, (QB, 1), 0)
    colsNB = jax.lax.broadcasted_iota(jnp.int32, (QB, NB), 1)
    cmask = row_s >= (colsNB * BS + (BS - 1))
    p = jax.lax.broadcasted_iota(jnp.int32, (QB, S), 1)
    causal = p <= row_s
    cbias = jnp.where(causal, -BIG, -2.0 * BIG)
    WB = QB + WIN
    wstart = jnp.maximum(base - WIN, 0)
    p_band = wstart + jax.lax.broadcasted_iota(jnp.int32, (QB, WB), 1)
    relb = row_s - p_band
    wmask = (relb >= 0) & (relb < WIN)

    gz = jnp.dot(hid_ref[...], scw_ref[...],
                 preferred_element_type=jnp.float32) + scb_ref[...]
    gates = jax.nn.sigmoid(gz)

    acc = jnp.zeros((QB, chw_ref.shape[1]), jnp.float32)

    for hk in range(HK):
        k = k_ref[hk]
        v = v_ref[hk]
        ck = ck_ref[hk]
        cv = cv_ref[hk]
        memk = memk_ref[hk:hk + 1]
        memv = memv_ref[hk:hk + 1]

        imp3 = None
        out_cs = []
        for g in range(G):
            qg = q_ref[hk, g] * scale
            simc = jnp.dot(qg, ck.T, preferred_element_type=jnp.float32)
            simm = jnp.dot(qg, memk.T, preferred_element_type=jnp.float32)
            simc = jnp.where(cmask, simc, -1e9)
            m = jnp.maximum(jnp.max(simc, axis=-1, keepdims=True), simm)
            eb = jnp.exp(simc - m)
            em = jnp.exp(simm - m)
            rden = 1.0 / (jnp.sum(eb, axis=-1, keepdims=True) + em)
            attn_b = eb * rden
            out_c = (jnp.dot(attn_b, cv, preferred_element_type=jnp.float32)
                     + (em * rden) * memv)
            imp3 = attn_b if imp3 is None else imp3 + attn_b
            out_cs.append(out_c)

        vals = jnp.where(cmask, imp3 * (1.0 / 3.0), -1.0)
        oh = None
        for _ in range(NUM_SEL):
            mx = jnp.max(vals, axis=-1, keepdims=True)
            idx = jnp.min(jnp.where(vals == mx, colsNB, NB),
                          axis=-1, keepdims=True)
            oh_t = (colsNB == idx)
            oh = oh_t if oh is None else oh | oh_t
            vals = jnp.where(oh_t, -2.0, vals)
        ohs = jnp.where(oh, BIG, 0.0).astype(jnp.bfloat16)
        bias0 = jnp.dot(ohs, exp_ref[...], preferred_element_type=jnp.float32)
        fbias = bias0 + cbias

        k_bf = k.astype(jnp.bfloat16)
        v_bf = v.astype(jnp.bfloat16)
        k_band = k_ref[hk, pl.ds(wstart, WB), :].astype(jnp.bfloat16)
        v_band = v_ref[hk, pl.ds(wstart, WB), :].astype(jnp.bfloat16)
        for g in range(G):
            qg_bf = (q_ref[hk, g] * scale).astype(jnp.bfloat16)
            h = hk * G + g
            g_c = gates[:, 3 * h:3 * h + 1]
            g_f = gates[:, 3 * h + 1:3 * h + 2]
            g_w = gates[:, 3 * h + 2:3 * h + 3]
            sim = jnp.dot(qg_bf, k_bf.T, preferred_element_type=jnp.float32)
            sf = sim + fbias
            mf = jnp.max(sf, axis=-1, keepdims=True)
            ef = jnp.exp(sf - mf)
            sum_f = jnp.sum(ef, axis=-1, keepdims=True)
            simw = jnp.dot(qg_bf, k_band.T, preferred_element_type=jnp.float32)
            sw = jnp.where(wmask, simw, -1e9)
            mw = jnp.max(sw, axis=-1, keepdims=True)
            ew = jnp.exp(sw - mw)
            sum_w = jnp.sum(ew, axis=-1, keepdims=True)
            out_f = jnp.dot(ef.astype(jnp.bfloat16), v_bf,
                            preferred_element_type=jnp.float32)
            out_w = jnp.dot(ew.astype(jnp.bfloat16), v_band,
                            preferred_element_type=jnp.float32)
            comb = ((g_f / sum_f) * out_f
                    + (g_w / sum_w) * out_w
                    + g_c * out_cs[g])
            acc = acc + jnp.dot(comb.astype(jnp.bfloat16),
                                chw_ref[h * D:(h + 1) * D, :],
                                preferred_element_type=jnp.float32)

    out_ref[...] = acc


def kernel(hidden_states, q, k, v, compress_mem_kv, k_norm_w, v_norm_w,
           sc_w, sc_b, ch_w):
    B, H, S, D = q.shape
    HK = k.shape[1]
    G = H // HK
    BS = 16
    NUM_SEL = 4
    WIN = 64
    NB = S // BS
    scale = D ** -0.5
    QB = 256
    hidden = H * D

    q4 = q[0].reshape(HK, G, S, D)
    k3 = k[0]
    v3 = v[0]
    memk = compress_mem_kv[0, :, 0, :]
    memv = compress_mem_kv[1, :, 0, :]
    knw = k_norm_w.reshape(1, D)
    vnw = v_norm_w.reshape(1, D)
    scw_t = sc_w.T
    scb = sc_b.reshape(1, 3 * H)
    chw_t = ch_w.T.astype(jnp.bfloat16)
    hid = hidden_states[0]
    expander = (jnp.arange(NB)[:, None] == jnp.arange(S)[None, :] // BS
                ).astype(jnp.bfloat16)

    ck_all, cv_all = pl.pallas_call(
        functools.partial(_compress_kernel, NB=NB, BS=BS, D=D),
        grid=(HK,),
        in_specs=[
            pl.BlockSpec((HK, S, D), lambda i: (0, 0, 0)),
            pl.BlockSpec((HK, S, D), lambda i: (0, 0, 0)),
            pl.BlockSpec((1, D), lambda i: (0, 0)),
            pl.BlockSpec((1, D), lambda i: (0, 0)),
        ],
        out_specs=[
            pl.BlockSpec((1, NB, D), lambda i: (i, 0, 0)),
            pl.BlockSpec((1, NB, D), lambda i: (i, 0, 0)),
        ],
        out_shape=[
            jax.ShapeDtypeStruct((HK, NB, D), jnp.float32),
            jax.ShapeDtypeStruct((HK, NB, D), jnp.float32),
        ],
    )(k3, v3, knw, vnw)

    def _segment(qoff, nq, SK, prev):
        kfn = functools.partial(_fused_kernel, QB=QB, S=SK, HK=HK, G=G, D=D,
                                BS=BS, NB=NB, NUM_SEL=NUM_SEL, WIN=WIN,
                                scale=scale, QOFF=qoff)
        return pl.pallas_call(
            kfn,
            grid=(nq,),
            in_specs=[
                pl.BlockSpec((HK, G, QB, D),
                             lambda i: (0, 0, i + qoff, 0)),
                pl.BlockSpec((HK, SK, D), lambda i: (0, 0, 0)),
                pl.BlockSpec((HK, SK, D), lambda i: (0, 0, 0)),
                pl.BlockSpec((HK, NB, D), lambda i: (0, 0, 0)),
                pl.BlockSpec((HK, NB, D), lambda i: (0, 0, 0)),
                pl.BlockSpec((HK, D), lambda i: (0, 0)),
                pl.BlockSpec((HK, D), lambda i: (0, 0)),
                pl.BlockSpec((hidden, 3 * H), lambda i: (0, 0)),
                pl.BlockSpec((1, 3 * H), lambda i: (0, 0)),
                pl.BlockSpec((QB, hidden), lambda i: (i + qoff, 0)),
                pl.BlockSpec((hidden, hidden), lambda i: (0, 0)),
                pl.BlockSpec((NB, SK), lambda i: (0, 0)),
                pl.BlockSpec((QB, hidden), lambda i: (i + qoff, 0)),
            ],
            out_specs=pl.BlockSpec((QB, hidden), lambda i: (i + qoff, 0)),
            out_shape=jax.ShapeDtypeStruct((S, hidden), jnp.float32),
            input_output_aliases={12: 0},
            compiler_params=pltpu.CompilerParams(
                dimension_semantics=("parallel",)),
        )(q4, k3, v3, ck_all, cv_all, memk, memv, scw_t, scb, hid, chw_t,
          expander, prev)

    half = S // (2 * QB)
    out0 = jnp.zeros((S, hidden), jnp.float32)
    out1 = _segment(0, half, S // 2, out0)
    out = _segment(half, half, S, out1)
    return out.reshape(B, S, hidden)


# speedup vs baseline: 1.0116x; 1.0116x over previous
import functools

import jax
import jax.numpy as jnp
from jax.experimental import pallas as pl
from jax.experimental.pallas import tpu as pltpu


def _compress_kernel(k_ref, v_ref, knw_ref, vnw_ref, ck_ref, cv_ref,
                     *, NB, BS, D):
    hk = pl.program_id(0)
    kb = k_ref[hk].reshape(NB, BS, D).mean(axis=1)
    vb = v_ref[hk].reshape(NB, BS, D).mean(axis=1)
    ck_ref[0] = kb * jax.lax.rsqrt(
        jnp.mean(kb * kb, axis=-1, keepdims=True) + 1e-6) * knw_ref[...]
    cv_ref[0] = vb * jax.lax.rsqrt(
        jnp.mean(vb * vb, axis=-1, keepdims=True) + 1e-6) * vnw_ref[...]


def _fused_kernel(q_ref, k_ref, v_ref, ck_ref, cv_ref, memk_ref, memv_ref,
                  scw_ref, scb_ref, hid_ref, chw_ref, exp_ref, out_ref,
                  *, QB, S, HK, G, D, BS, NB, NUM_SEL, WIN, scale, QOFF):
    qi = pl.program_id(0)
    base = (QOFF + qi) * QB
    BIG = jnp.float32(2.0 ** 30)
    row_s = base + jax.lax.broadcasted_iota(jnp.int32, (QB, 1), 0)
    colsNB = jax.lax.broadcasted_iota(jnp.int32, (QB, NB), 1)
    cmask = row_s >= (colsNB * BS + (BS - 1))
    p = jax.lax.broadcasted_iota(jnp.int32, (QB, S), 1)
    causal = p <= row_s
    cbias = jnp.where(causal, -BIG, -2.0 * BIG)
    WB = QB + WIN
    wstart = jnp.maximum(base - WIN, 0)
    p_band = wstart + jax.lax.broadcasted_iota(jnp.int32, (QB, WB), 1)
    relb = row_s - p_band
    wmask = (relb >= 0) & (relb < WIN)

    gz = jnp.dot(hid_ref[...], scw_ref[...],
                 preferred_element_type=jnp.float32) + scb_ref[...]
    gates = jax.nn.sigmoid(gz)

    acc = jnp.zeros((QB, chw_ref.shape[1]), jnp.float32)

    for hk in range(HK):
        k = k_ref[hk]
        v = v_ref[hk]
        ck = ck_ref[hk]
        cv = cv_ref[hk]
        memk = memk_ref[hk:hk + 1]
        memv = memv_ref[hk:hk + 1]

        imp3 = None
        out_cs = []
        for g in range(G):
            qg = q_ref[hk, g] * scale
            simc = jnp.dot(qg, ck.T, preferred_element_type=jnp.float32)
            simm = jnp.dot(qg, memk.T, preferred_element_type=jnp.float32)
            simc = jnp.where(cmask, simc, -1e9)
            m = jnp.maximum(jnp.max(simc, axis=-1, keepdims=True), simm)
            eb = jnp.exp(simc - m)
            em = jnp.exp(simm - m)
            rden = 1.0 / (jnp.sum(eb, axis=-1, keepdims=True) + em)
            attn_b = eb * rden
            out_c = (jnp.dot(attn_b, cv, preferred_element_type=jnp.float32)
                     + (em * rden) * memv)
            imp3 = attn_b if imp3 is None else imp3 + attn_b
            out_cs.append(out_c)

        vals = jnp.where(cmask, imp3 * (1.0 / 3.0), -1.0)
        oh = None
        for _ in range(NUM_SEL):
            mx = jnp.max(vals, axis=-1, keepdims=True)
            idx = jnp.min(jnp.where(vals == mx, colsNB, NB),
                          axis=-1, keepdims=True)
            oh_t = (colsNB == idx)
            oh = oh_t if oh is None else oh | oh_t
            vals = jnp.where(oh_t, -2.0, vals)
        ohs = jnp.where(oh, BIG, 0.0).astype(jnp.bfloat16)
        bias0 = jnp.dot(ohs, exp_ref[...], preferred_element_type=jnp.float32)
        fbias = bias0 + cbias

        k_bf = k.astype(jnp.bfloat16)
        v_bf = v.astype(jnp.bfloat16)
        k_band = k_ref[hk, pl.ds(wstart, WB), :].astype(jnp.bfloat16)
        v_band = v_ref[hk, pl.ds(wstart, WB), :].astype(jnp.bfloat16)
        for g in range(G):
            qg_bf = (q_ref[hk, g] * scale).astype(jnp.bfloat16)
            h = hk * G + g
            g_c = gates[:, 3 * h:3 * h + 1]
            g_f = gates[:, 3 * h + 1:3 * h + 2]
            g_w = gates[:, 3 * h + 2:3 * h + 3]
            sim = jnp.dot(qg_bf, k_bf.T, preferred_element_type=jnp.float32)
            sf = sim + fbias
            mf = jnp.max(sf, axis=-1, keepdims=True)
            ef = jnp.exp(sf - mf)
            sum_f = jnp.sum(ef, axis=-1, keepdims=True)
            simw = jnp.dot(qg_bf, k_band.T, preferred_element_type=jnp.float32)
            sw = jnp.where(wmask, simw, -1e9)
            mw = jnp.max(sw, axis=-1, keepdims=True)
            ew = jnp.exp(sw - mw)
            sum_w = jnp.sum(ew, axis=-1, keepdims=True)
            out_f = jnp.dot(ef.astype(jnp.bfloat16), v_bf,
                            preferred_element_type=jnp.float32)
            out_w = jnp.dot(ew.astype(jnp.bfloat16), v_band,
                            preferred_element_type=jnp.float32)
            comb = ((g_f / sum_f) * out_f
                    + (g_w / sum_w) * out_w
                    + g_c * out_cs[g])
            acc = acc + jnp.dot(comb.astype(jnp.bfloat16),
                                chw_ref[h * D:(h + 1) * D, :],
                                preferred_element_type=jnp.float32)

    out_ref[...] = acc


def kernel(hidden_states, q, k, v, compress_mem_kv, k_norm_w, v_norm_w,
           sc_w, sc_b, ch_w):
    B, H, S, D = q.shape
    HK = k.shape[1]
    G = H // HK
    BS = 16
    NUM_SEL = 4
    WIN = 64
    NB = S // BS
    scale = D ** -0.5
    QB = 256
    hidden = H * D

    q4 = q[0].reshape(HK, G, S, D)
    k3 = k[0]
    v3 = v[0]
    memk = compress_mem_kv[0, :, 0, :]
    memv = compress_mem_kv[1, :, 0, :]
    knw = k_norm_w.reshape(1, D)
    vnw = v_norm_w.reshape(1, D)
    scw_t = sc_w.T
    scb = sc_b.reshape(1, 3 * H)
    chw_t = ch_w.T.astype(jnp.bfloat16)
    hid = hidden_states[0]
    expander = (jnp.arange(NB)[:, None] == jnp.arange(S)[None, :] // BS
                ).astype(jnp.bfloat16)

    ck_all, cv_all = pl.pallas_call(
        functools.partial(_compress_kernel, NB=NB, BS=BS, D=D),
        grid=(HK,),
        in_specs=[
            pl.BlockSpec((HK, S, D), lambda i: (0, 0, 0)),
            pl.BlockSpec((HK, S, D), lambda i: (0, 0, 0)),
            pl.BlockSpec((1, D), lambda i: (0, 0)),
            pl.BlockSpec((1, D), lambda i: (0, 0)),
        ],
        out_specs=[
            pl.BlockSpec((1, NB, D), lambda i: (i, 0, 0)),
            pl.BlockSpec((1, NB, D), lambda i: (i, 0, 0)),
        ],
        out_shape=[
            jax.ShapeDtypeStruct((HK, NB, D), jnp.float32),
            jax.ShapeDtypeStruct((HK, NB, D), jnp.float32),
        ],
    )(k3, v3, knw, vnw)

    def _segment(qoff, nq, SK):
        kfn = functools.partial(_fused_kernel, QB=QB, S=SK, HK=HK, G=G, D=D,
                                BS=BS, NB=NB, NUM_SEL=NUM_SEL, WIN=WIN,
                                scale=scale, QOFF=qoff)
        return pl.pallas_call(
            kfn,
            grid=(nq,),
            in_specs=[
                pl.BlockSpec((HK, G, QB, D),
                             lambda i: (0, 0, i + qoff, 0)),
                pl.BlockSpec((HK, SK, D), lambda i: (0, 0, 0)),
                pl.BlockSpec((HK, SK, D), lambda i: (0, 0, 0)),
                pl.BlockSpec((HK, NB, D), lambda i: (0, 0, 0)),
                pl.BlockSpec((HK, NB, D), lambda i: (0, 0, 0)),
                pl.BlockSpec((HK, D), lambda i: (0, 0)),
                pl.BlockSpec((HK, D), lambda i: (0, 0)),
                pl.BlockSpec((hidden, 3 * H), lambda i: (0, 0)),
                pl.BlockSpec((1, 3 * H), lambda i: (0, 0)),
                pl.BlockSpec((QB, hidden), lambda i: (i + qoff, 0)),
                pl.BlockSpec((hidden, hidden), lambda i: (0, 0)),
                pl.BlockSpec((NB, SK), lambda i: (0, 0)),
            ],
            out_specs=pl.BlockSpec((QB, hidden), lambda i: (i, 0)),
            out_shape=jax.ShapeDtypeStruct((nq * QB, hidden), jnp.float32),
            compiler_params=pltpu.CompilerParams(
                dimension_semantics=("parallel",)),
        )(q4, k3, v3, ck_all, cv_all, memk, memv, scw_t, scb, hid, chw_t,
          expander)

    half = S // (2 * QB)
    out = jnp.concatenate(
        [_segment(0, half, S // 2), _segment(half, half, S)], axis=0)
    return out.reshape(B, S, hidden)
